# baseline (speedup 1.0000x reference)
import functools

import jax
import jax.numpy as jnp
from jax import lax
from jax.experimental import pallas as pl
from jax.experimental.pallas import tpu as pltpu
from jax.experimental.pallas import tpu_sc as plsc

BATCH = 16384
EMB = 32
NC = 2
NS = 16
NW = NC * NS
BPW = BATCH // NW
CHUNK = 128
NCHUNK = BPW // CHUNK

_sc_mesh = plsc.VectorSubcoreMesh(core_axis_name="c", subcore_axis_name="s")


@functools.partial(
    pl.kernel,
    out_type=(
        jax.ShapeDtypeStruct((BATCH, EMB), jnp.float32),
        jax.ShapeDtypeStruct((BATCH, EMB), jnp.float32),
        jax.ShapeDtypeStruct((BATCH, EMB), jnp.float32),
    ),
    mesh=_sc_mesh,
    scratch_types=(
        pltpu.VMEM((BPW,), jnp.int32),
        pltpu.VMEM((BPW,), jnp.int32),
        pltpu.VMEM((BPW, EMB), jnp.float32),
        pltpu.VMEM((BPW, EMB), jnp.float32),
        pltpu.VMEM((BPW, EMB), jnp.float32),
        pltpu.VMEM((BPW, EMB), jnp.float32),
        pltpu.SemaphoreType.DMA,
        pltpu.SemaphoreType.DMA,
    ),
    compiler_params=pltpu.CompilerParams(use_tc_tiling_on_sc=False),
)
def _sc_gather(user_hbm, item_hbm, gu_hbm, gi_hbm, mu_hbm, mi_hbm,
               out_gmf, out_mu, out_mi,
               idx_u, idx_i, b_gu, b_gi, b_mu, b_mi, sem_g, sem_m):
    wid = lax.axis_index("s") * NC + lax.axis_index("c")
    base = wid * BPW
    pltpu.sync_copy(user_hbm.at[pl.ds(base, BPW)], idx_u)
    pltpu.sync_copy(item_hbm.at[pl.ds(base, BPW)], idx_i)
    gmf_copies = []
    mlp_copies = []
    for c in range(NCHUNK):
        s = pl.ds(c * CHUNK, CHUNK)
        gmf_copies.append(pltpu.async_copy(gu_hbm.at[idx_u.at[s]], b_gu.at[s], sem_g))
        gmf_copies.append(pltpu.async_copy(gi_hbm.at[idx_i.at[s]], b_gi.at[s], sem_g))
    for c in range(NCHUNK):
        s = pl.ds(c * CHUNK, CHUNK)
        mlp_copies.append(pltpu.async_copy(mu_hbm.at[idx_u.at[s]], b_mu.at[s], sem_m))
        mlp_copies.append(pltpu.async_copy(mi_hbm.at[idx_i.at[s]], b_mi.at[s], sem_m))
    for cp in gmf_copies:
        cp.wait()

    def mul_body(r, carry):
        for off in (0, 16):
            sl = pl.ds(off, 16)
            b_gu[r, sl] = b_gu[r, sl] * b_gi[r, sl]
        return carry

    lax.fori_loop(0, BPW, mul_body, 0)
    st1 = pltpu.async_copy(b_gu, out_gmf.at[pl.ds(base, BPW)], sem_g)
    for cp in mlp_copies:
        cp.wait()
    st2 = pltpu.async_copy(b_mu, out_mu.at[pl.ds(base, BPW)], sem_m)
    st3 = pltpu.async_copy(b_mi, out_mi.at[pl.ds(base, BPW)], sem_m)
    st1.wait()
    st2.wait()
    st3.wait()


def _mlp_body(gmf_ref, mu_ref, mi_ref, w1a, w1b, b1, w2, b2, w3, b3,
              wog, woh, bo, out_ref):
    f32 = jnp.float32
    h = jnp.dot(mu_ref[...], w1a[...], preferred_element_type=f32)
    h = h + jnp.dot(mi_ref[...], w1b[...], preferred_element_type=f32)
    h = jnp.maximum(h + b1[...], 0.0)
    h = jnp.maximum(jnp.dot(h, w2[...], preferred_element_type=f32) + b2[...], 0.0)
    h = jnp.maximum(jnp.dot(h, w3[...], preferred_element_type=f32) + b3[...], 0.0)
    logit = jnp.dot(gmf_ref[...], wog[...], preferred_element_type=f32)
    logit = logit + jnp.dot(h, woh[...], preferred_element_type=f32) + bo[0]
    out_ref[...] = jax.nn.sigmoid(logit)


_mlp = pl.pallas_call(
    _mlp_body,
    out_shape=jax.ShapeDtypeStruct((BATCH, 1), jnp.float32),
)


def kernel(user, item, gmf_user_emb, gmf_item_emb, mlp_user_emb, mlp_item_emb,
           W1, b1, W2, b2, W3, b3, Wo, bo):
    gmf, mu, mi = _sc_gather(user, item, gmf_user_emb, gmf_item_emb,
                             mlp_user_emb, mlp_item_emb)
    out = _mlp(gmf, mu, mi, W1[:EMB], W1[EMB:], b1, W2, b2, W3, b3,
               Wo[:EMB], Wo[EMB:], bo)
    return out[:, 0]
